# initial kernel scaffold (unmeasured)
import jax
import jax.numpy as jnp
from jax import lax
from jax.experimental import pallas as pl
from jax.experimental.pallas import tpu as pltpu

S = 2048
K = 4096
N = 8192
HALF = S // 2
NT = 16
NTS = N // NT


def kernel(O, Wo):
    A = O.reshape(S, K).astype(jnp.bfloat16)
    x = lax.axis_index("x")
    xs = jnp.reshape(x.astype(jnp.int32), (1,))

    def body(s_ref, a_ref, w_ref, out_ref, recv_ref,
             send_buf, recv_vmem, send_sems, recv_sems, local_sem):
        h = pl.program_id(0)
        n = pl.program_id(1)
        my_x = s_ref[0]
        peer = (1 - my_x, lax.axis_index("y"), lax.axis_index("z"))

        @pl.when(jnp.logical_and(h == 0, n == 0))
        def _():
            barrier = pltpu.get_barrier_semaphore()
            pl.semaphore_signal(barrier, inc=1, device_id=peer,
                                device_id_type=pl.DeviceIdType.MESH)
            pl.semaphore_wait(barrier, 1)

        acc = jnp.dot(a_ref[...], w_ref[...].astype(jnp.bfloat16),
                      preferred_element_type=jnp.float32)

        def send_desc(slot, nn):
            return pltpu.make_async_remote_copy(
                src_ref=send_buf.at[slot],
                dst_ref=recv_ref.at[:, pl.ds(nn * NTS, NTS)],
                send_sem=send_sems.at[slot],
                recv_sem=recv_sems.at[nn],
                device_id=peer,
                device_id_type=pl.DeviceIdType.MESH,
            )

        @pl.when(h == 0)
        def _():
            for nn in range(NT):
                @pl.when(n == nn)
                def _(nn=nn):
                    slot = nn % 2
                    if nn >= 2:
                        send_desc(slot, nn - 2).wait_send()
                    send_buf[slot, :, :] = acc
                    send_desc(slot, nn).start()

        @pl.when(h == 1)
        def _():
            for nn in range(NT):
                @pl.when(n == nn)
                def _(nn=nn):
                    if nn < 2:
                        send_desc(nn, NT - 2 + nn).wait_send()
                    send_desc(0, nn).wait_recv()
                    cp = pltpu.make_async_copy(
                        recv_ref.at[:, pl.ds(nn * NTS, NTS)],
                        recv_vmem, local_sem)
                    cp.start()
                    cp.wait()
                    out_ref[...] = acc + recv_vmem[...]

    grid_spec = pltpu.PrefetchScalarGridSpec(
        num_scalar_prefetch=1,
        grid=(2, NT),
        in_specs=[
            pl.BlockSpec(
                (HALF, K),
                lambda h, n, s: (jnp.where(h == 0, 1 - s[0], s[0]), 0)),
            pl.BlockSpec((K, NTS), lambda h, n, s: (0, n)),
        ],
        out_specs=[
            pl.BlockSpec(
                (HALF, NTS),
                lambda h, n, s: (0, jnp.where(h == 0, 0, n))),
            pl.BlockSpec(memory_space=pltpu.ANY),
        ],
        scratch_shapes=[
            pltpu.VMEM((2, HALF, NTS), jnp.float32),
            pltpu.VMEM((HALF, NTS), jnp.float32),
            pltpu.SemaphoreType.DMA((2,)),
            pltpu.SemaphoreType.DMA((NT,)),
            pltpu.SemaphoreType.DMA,
        ],
    )
    out, _ = pl.pallas_call(
        body,
        grid_spec=grid_spec,
        out_shape=[
            jax.ShapeDtypeStruct((HALF, N), jnp.float32),
            jax.ShapeDtypeStruct((HALF, N), jnp.float32),
        ],
        compiler_params=pltpu.CompilerParams(
            collective_id=0,
            dimension_semantics=("arbitrary", "arbitrary"),
        ),
    )(xs, A, Wo)
    return out.reshape(1, HALF, N)


# baseline (device time: 521854 ns/iter reference)
import jax
import jax.numpy as jnp
from jax import lax
from jax.experimental import pallas as pl
from jax.experimental.pallas import tpu as pltpu

S = 2048
K = 4096
N = 8192
HALF = S // 2
NT = 16
NTS = N // NT


def kernel(O, Wo):
    A = O.reshape(S, K).astype(jnp.bfloat16)
    x = lax.axis_index("x")
    xs = jnp.reshape(x.astype(jnp.int32), (1,))

    def body(s_ref, a_ref, w_ref, out_ref, recv_ref,
             send_buf, recv_vmem, send_sems, recv_sems, local_sem):
        h = pl.program_id(0)
        n = pl.program_id(1)
        my_x = s_ref[0]
        peer = (1 - my_x, lax.axis_index("y"), lax.axis_index("z"))

        @pl.when(jnp.logical_and(h == 0, n == 0))
        def _():
            barrier = pltpu.get_barrier_semaphore()
            pl.semaphore_signal(barrier, inc=1, device_id=peer,
                                device_id_type=pl.DeviceIdType.MESH)
            pl.semaphore_wait(barrier, 1)

        acc = jnp.dot(a_ref[...], w_ref[...].astype(jnp.bfloat16),
                      preferred_element_type=jnp.float32)

        def send_desc(slot, nn):
            return pltpu.make_async_remote_copy(
                src_ref=send_buf.at[slot],
                dst_ref=recv_ref.at[:, pl.ds(nn * NTS, NTS)],
                send_sem=send_sems.at[slot],
                recv_sem=recv_sems.at[nn],
                device_id=peer,
                device_id_type=pl.DeviceIdType.MESH,
            )

        @pl.when(h == 0)
        def _():
            for nn in range(NT):
                @pl.when(n == nn)
                def _(nn=nn):
                    slot = nn % 2
                    if nn >= 2:
                        send_desc(slot, nn - 2).wait_send()
                    send_buf[slot, :, :] = acc
                    send_desc(slot, nn).start()

        @pl.when(h == 1)
        def _():
            for nn in range(NT):
                @pl.when(n == nn)
                def _(nn=nn):
                    if nn < 2:
                        send_desc(nn, NT - 2 + nn).wait_send()
                    send_desc(0, nn).wait_recv()
                    cp = pltpu.make_async_copy(
                        recv_ref.at[:, pl.ds(nn * NTS, NTS)],
                        recv_vmem, local_sem)
                    cp.start()
                    cp.wait()
                    out_ref[...] = acc + recv_vmem[...]

    grid_spec = pltpu.PrefetchScalarGridSpec(
        num_scalar_prefetch=1,
        grid=(2, NT),
        in_specs=[
            pl.BlockSpec(
                (HALF, K),
                lambda h, n, s: (jnp.where(h == 0, 1 - s[0], s[0]), 0)),
            pl.BlockSpec((K, NTS), lambda h, n, s: (0, n)),
        ],
        out_specs=[
            pl.BlockSpec(
                (HALF, NTS),
                lambda h, n, s: (0, jnp.where(h == 0, 0, n))),
            pl.BlockSpec(memory_space=pl.ANY),
        ],
        scratch_shapes=[
            pltpu.VMEM((2, HALF, NTS), jnp.float32),
            pltpu.VMEM((HALF, NTS), jnp.float32),
            pltpu.SemaphoreType.DMA((2,)),
            pltpu.SemaphoreType.DMA((NT,)),
            pltpu.SemaphoreType.DMA,
        ],
    )
    out, _ = pl.pallas_call(
        body,
        grid_spec=grid_spec,
        out_shape=[
            jax.ShapeDtypeStruct((HALF, N), jnp.float32),
            jax.ShapeDtypeStruct((HALF, N), jnp.float32),
        ],
        compiler_params=pltpu.CompilerParams(
            collective_id=0,
            dimension_semantics=("arbitrary", "arbitrary"),
        ),
    )(xs, A, Wo)
    return out.reshape(1, HALF, N)
